# baseline (speedup 1.0000x reference)
import functools
import math

import jax
import jax.numpy as jnp
from jax import lax
from jax.experimental import pallas as pl
from jax.experimental.pallas import tpu as pltpu
from jax.experimental.pallas import tpu_sc as plsc

_B = 2
_N = 2048
_DM = 1024
_H = 16
_DK = 64
_KK = 12
_NT = _B * _N

_NC = 2
_NS = 16
_NWRK = _NC * _NS
_TPW = _NT // _NWRK
_CH = 16
_NCH = _TPW // _CH


def _proj_body(x_ref, wq_ref, wk_ref, wv_ref, oq_ref, ok_ref, ov_ref):
    x = x_ref[...]
    oq_ref[...] = jnp.dot(x, wq_ref[...], preferred_element_type=jnp.float32)
    ok_ref[...] = jnp.dot(x, wk_ref[...], preferred_element_type=jnp.float32)
    ov_ref[...] = jnp.dot(x, wv_ref[...], preferred_element_type=jnp.float32)


def _projections(x, wq_in, wk_in, wv_in):
    tm = 256
    grid = (_NT // tm,)
    return pl.pallas_call(
        _proj_body,
        grid=grid,
        in_specs=[
            pl.BlockSpec((tm, _DM), lambda i: (i, 0)),
            pl.BlockSpec((_DM, _DM), lambda i: (0, 0)),
            pl.BlockSpec((_DM, _DM), lambda i: (0, 0)),
            pl.BlockSpec((_DM, _DM), lambda i: (0, 0)),
        ],
        out_specs=[
            pl.BlockSpec((tm, _DM), lambda i: (i, 0)),
            pl.BlockSpec((tm, _DM), lambda i: (i, 0)),
            pl.BlockSpec((tm, _DM), lambda i: (i, 0)),
        ],
        out_shape=[jax.ShapeDtypeStruct((_NT, _DM), jnp.float32)] * 3,
    )(x, wq_in, wk_in, wv_in)


def _mm_body(x_ref, w_ref, o_ref):
    o_ref[...] = jnp.dot(x_ref[...], w_ref[...], preferred_element_type=jnp.float32)


def _matmul(x, w):
    tm = 256
    n_rows = x.shape[0]
    return pl.pallas_call(
        _mm_body,
        grid=(n_rows // tm,),
        in_specs=[
            pl.BlockSpec((tm, _DM), lambda i: (i, 0)),
            pl.BlockSpec((_DM, _DM), lambda i: (0, 0)),
        ],
        out_specs=pl.BlockSpec((tm, _DM), lambda i: (i, 0)),
        out_shape=jax.ShapeDtypeStruct((n_rows, _DM), jnp.float32),
    )(x, w)


def _global_body(k_ref, v_ref, qg_ref, p_ref, pt_ref, o_ref):
    qrow = qg_ref[0]
    a = k_ref[0] * qrow
    logits = jnp.dot(a, p_ref[...], preferred_element_type=jnp.float32)
    m = jnp.max(logits, axis=0, keepdims=True)
    e = jnp.exp(logits - m)
    s = jnp.sum(e, axis=0, keepdims=True)
    prob = e / s
    pe = jnp.dot(prob, pt_ref[...], preferred_element_type=jnp.float32)
    o_ref[0, 0, :] = jnp.sum(pe * v_ref[0], axis=0)


def _global_attention(kt, vt, qg, p, pt):
    return pl.pallas_call(
        _global_body,
        grid=(4,),
        in_specs=[
            pl.BlockSpec((1, _N, _DM), lambda g: (g // 2, 0, 0)),
            pl.BlockSpec((1, _N, _DM), lambda g: (g // 2, 0, 0)),
            pl.BlockSpec((1, 1, _DM), lambda g: (g, 0, 0)),
            pl.BlockSpec((_DM, _H), lambda g: (0, 0)),
            pl.BlockSpec((_H, _DM), lambda g: (0, 0)),
        ],
        out_specs=pl.BlockSpec((1, 1, _DM), lambda g: (g, 0, 0)),
        out_shape=jax.ShapeDtypeStruct((4, 1, _DM), jnp.float32),
    )(kt, vt, qg, p, pt)


_PAIR = 2
_R = 8
_PR = _PAIR * _R
_WROWS = _CH + 2


def _sc_body(qt_hbm, kt_hbm, vt_hbm, idx_hbm, out_hbm,
             idxv, qv, krand, vrand, kwin, vwin, kgv, vgv, outv,
             semk, semv, semw):
    wid = lax.axis_index("s") * _NC + lax.axis_index("c")

    npairs = _TPW // _PAIR

    gbase = pl.multiple_of((wid // _NS) * _N, _N)
    pltpu.sync_copy(kt_hbm.at[pl.ds(gbase, 1)], kgv)
    pltpu.sync_copy(vt_hbm.at[pl.ds(gbase, 1)], vgv)
    pltpu.sync_copy(idx_hbm.at[pl.ds(pl.multiple_of(wid * npairs, 8),
                                     npairs)], idxv)
    pltpu.async_copy(kt_hbm.at[idxv.at[0]], krand, semk)

    def chunk_body(c, carry):
        base = pl.multiple_of(wid * _TPW + c * _CH, _CH)
        pltpu.sync_copy(qt_hbm.at[pl.ds(base, _CH)], qv)
        wstart = jnp.clip(base - 1, 0, _NT - _WROWS)
        roff = base - 1 - wstart
        cw = pltpu.async_copy(kt_hbm.at[pl.ds(wstart, _WROWS)], kwin, semw)
        cw2 = pltpu.async_copy(vt_hbm.at[pl.ds(wstart, _WROWS)], vwin, semw)
        cw.wait()
        cw2.wait()

        def pair_body(p, carry2):
            gp = c * (_CH // _PAIR) + p
            pltpu.make_async_copy(kt_hbm.at[idxv.at[gp]], krand, semk).wait()
            cv = pltpu.async_copy(vt_hbm.at[idxv.at[gp]], vrand, semv)

            t0 = p * _PAIR
            rel0 = jnp.clip(t0 + roff, 0, _CH - 1)
            rel1 = jnp.clip(t0 + 1 + roff, 0, _CH - 1)

            def d_body(j, accs):
                new = list(accs)
                for l in range(8):
                    dsl = pl.ds(l * 16, 16)
                    qd0 = qv[t0, j, dsl]
                    qd1 = qv[t0 + 1, j, dsl]
                    kg = kgv[0, j, dsl]
                    new[0] = new[0] + qd0 * kg
                    new[_KK] = new[_KK] + qd1 * kg
                    for r in range(_R):
                        kr0 = krand[r, j, dsl]
                        kr1 = krand[_R + r, j, dsl]
                        new[1 + r] = new[1 + r] + qd0 * kr0
                        new[_KK + 1 + r] = new[_KK + 1 + r] + qd1 * kr1
                    for w in range(3):
                        kw0 = kwin[rel0 + w, j, dsl]
                        kw1 = kwin[rel1 + w, j, dsl]
                        new[9 + w] = new[9 + w] + qd0 * kw0
                        new[_KK + 9 + w] = new[_KK + 9 + w] + qd1 * kw1
                return tuple(new)

            zero = jnp.zeros((16,), jnp.float32)
            accs = lax.fori_loop(0, 8, d_body, (zero,) * (2 * _KK))

            all_ws = []
            for u in range(_PAIR):
                ko = u * _KK
                m = accs[ko]
                for k in range(1, _KK):
                    m = jnp.maximum(m, accs[ko + k])
                es = tuple(jnp.exp(accs[ko + k] - m) for k in range(_KK))
                s = es[0]
                for k in range(1, _KK):
                    s = s + es[k]
                inv = 1.0 / s
                all_ws.append(tuple(e * inv for e in es))

            cv.wait()
            nxt = jnp.minimum(gp + 1, npairs - 1)
            pltpu.async_copy(kt_hbm.at[idxv.at[nxt]], krand, semk)

            ws0, ws1 = all_ws

            def o_body(j, carry3):
                for l in range(8):
                    dsl = pl.ds(l * 16, 16)
                    vg = vgv[0, j, dsl]
                    acc0 = ws0[0] * vg
                    acc1 = ws1[0] * vg
                    for r in range(_R):
                        acc0 = acc0 + ws0[1 + r] * vrand[r, j, dsl]
                        acc1 = acc1 + ws1[1 + r] * vrand[_R + r, j, dsl]
                    for w in range(3):
                        acc0 = acc0 + ws0[9 + w] * vwin[rel0 + w, j, dsl]
                        acc1 = acc1 + ws1[9 + w] * vwin[rel1 + w, j, dsl]
                    outv[t0, j, dsl] = acc0
                    outv[t0 + 1, j, dsl] = acc1
                return carry3

            lax.fori_loop(0, 8, o_body, 0)
            return carry2

        lax.fori_loop(0, _CH // _PAIR, pair_body, 0)
        pltpu.sync_copy(outv, out_hbm.at[pl.ds(base, _CH)])
        return carry

    lax.fori_loop(0, _NCH, chunk_body, 0)
    pltpu.make_async_copy(kt_hbm.at[idxv.at[npairs - 1]], krand, semk).wait()


def _sc_attention(qt, kt, vt, idx_rand):
    mesh = plsc.VectorSubcoreMesh(core_axis_name="c", subcore_axis_name="s")
    qt3 = qt.reshape(_NT, 8, 128)
    kt3 = kt.reshape(_NT, 8, 128)
    vt3 = vt.reshape(_NT, 8, 128)
    fn = functools.partial(
        pl.kernel,
        mesh=mesh,
        out_type=jax.ShapeDtypeStruct((_NT, 8, 128), jnp.float32),
        scratch_types=[
            pltpu.VMEM((_TPW // _PAIR, _PR), jnp.int32),
            pltpu.VMEM((_CH, 8, 128), jnp.float32),
            pltpu.VMEM((_PR, 8, 128), jnp.float32),
            pltpu.VMEM((_PR, 8, 128), jnp.float32),
            pltpu.VMEM((_WROWS, 8, 128), jnp.float32),
            pltpu.VMEM((_WROWS, 8, 128), jnp.float32),
            pltpu.VMEM((1, 8, 128), jnp.float32),
            pltpu.VMEM((1, 8, 128), jnp.float32),
            pltpu.VMEM((_CH, 8, 128), jnp.float32),
            pltpu.SemaphoreType.DMA,
            pltpu.SemaphoreType.DMA,
            pltpu.SemaphoreType.DMA,
        ],
    )(_sc_body)
    out = fn(qt3, kt3, vt3, idx_rand.reshape(_NT // _PAIR, _PR))
    return out.reshape(_NT, _DM)


def _out_body(x_ref, g_ref, tgt_ref, w_ref, o_ref):
    x = x_ref[...]
    tgt = tgt_ref[pl.program_id(0)]
    rows = lax.broadcasted_iota(jnp.int32, x.shape, 0)
    x = jnp.where(rows == tgt, g_ref[0], x)
    o_ref[...] = jnp.dot(x, w_ref[...], preferred_element_type=jnp.float32)


def _out_matmul(x, g16, tgt, w):
    tm = 256
    return pl.pallas_call(
        _out_body,
        grid=(_NT // tm,),
        in_specs=[
            pl.BlockSpec((tm, _DM), lambda i: (i, 0)),
            pl.BlockSpec((1, 1, _DM), lambda i: (i, 0, 0)),
            pl.BlockSpec(memory_space=pltpu.SMEM),
            pl.BlockSpec((_DM, _DM), lambda i: (0, 0)),
        ],
        out_specs=pl.BlockSpec((tm, _DM), lambda i: (i, 0)),
        out_shape=jax.ShapeDtypeStruct((_NT, _DM), jnp.float32),
    )(x, g16, tgt, w)


def kernel(Q, K, V, Wq, Wk, Wv, Wo, idx):
    scale = 1.0 / math.sqrt(_DK)
    ct = jnp.arange(_DM)
    cols = (ct % _H) * _DK + ct // _H

    wq_in = jnp.transpose(Wq[cols, :] * scale)
    wk_in = jnp.transpose(Wk[cols, :])
    wv_in = jnp.transpose(Wv[cols, :])
    wo_in = jnp.transpose(Wo[:, cols])

    qt = _matmul(Q.reshape(_NT, _DM), wq_in)
    kt = _matmul(K.reshape(_NT, _DM), wk_in)
    vt = _matmul(V.reshape(_NT, _DM), wv_in)

    p = jax.nn.one_hot(ct % _H, _H, dtype=jnp.float32)
    pt = jnp.transpose(p)

    kt3 = kt.reshape(_B, _N, _DM)
    vt3 = vt.reshape(_B, _N, _DM)
    qg = qt.reshape(_B, _N, _DM)[:, jnp.array([0, _N - 1]), :].reshape(4, 1, _DM)
    out_g = _global_attention(kt3, vt3, qg, p, pt).reshape(_B, 2, _DM)

    idx32 = idx[:, 1:1 + _R].astype(jnp.int32)
    blocks = []
    for b in range(_B):
        pad = jnp.full((1, _R), b * _N, jnp.int32)
        blocks.append(jnp.concatenate([pad, idx32 + b * _N, pad], axis=0))
    idx_rand = jnp.concatenate(blocks, axis=0)

    sc_out = _sc_attention(qt, kt, vt, idx_rand)

    g16 = jnp.zeros((16, 1, _DM), jnp.float32)
    g16 = g16.at[0, 0].set(out_g[0, 0]).at[7, 0].set(out_g[0, 1])
    g16 = g16.at[8, 0].set(out_g[1, 0]).at[15, 0].set(out_g[1, 1])
    tgt = jnp.array([0] + [-1] * 6 + [255, 0] + [-1] * 6 + [255], jnp.int32)

    x = _out_matmul(sc_out, g16, tgt, wo_in)
    return x.reshape(_B, _N, _DM)


# speedup vs baseline: 1.2531x; 1.2531x over previous
import functools
import math

import jax
import jax.numpy as jnp
from jax import lax
from jax.experimental import pallas as pl
from jax.experimental.pallas import tpu as pltpu
from jax.experimental.pallas import tpu_sc as plsc

_B = 2
_N = 2048
_DM = 1024
_H = 16
_DK = 64
_KK = 12
_NT = _B * _N

_NC = 2
_NS = 16
_NWRK = _NC * _NS
_TPW = _NT // _NWRK
_CH = 16
_NCH = _TPW // _CH


def _proj_body(x_ref, wq_ref, wk_ref, wv_ref, oq_ref, ok_ref, ov_ref):
    x = x_ref[...]
    oq_ref[...] = jnp.dot(x, wq_ref[...], preferred_element_type=jnp.float32)
    ok_ref[...] = jnp.dot(x, wk_ref[...], preferred_element_type=jnp.float32)
    ov_ref[...] = jnp.dot(x, wv_ref[...], preferred_element_type=jnp.float32)


def _projections(x, wq_in, wk_in, wv_in):
    tm = 256
    grid = (_NT // tm,)
    return pl.pallas_call(
        _proj_body,
        grid=grid,
        in_specs=[
            pl.BlockSpec((tm, _DM), lambda i: (i, 0)),
            pl.BlockSpec((_DM, _DM), lambda i: (0, 0)),
            pl.BlockSpec((_DM, _DM), lambda i: (0, 0)),
            pl.BlockSpec((_DM, _DM), lambda i: (0, 0)),
        ],
        out_specs=[
            pl.BlockSpec((tm, _DM), lambda i: (i, 0)),
            pl.BlockSpec((tm, _DM), lambda i: (i, 0)),
            pl.BlockSpec((tm, _DM), lambda i: (i, 0)),
        ],
        out_shape=[jax.ShapeDtypeStruct((_NT, _DM), jnp.float32)] * 3,
    )(x, wq_in, wk_in, wv_in)


def _mm_body(x_ref, w_ref, o_ref):
    o_ref[...] = jnp.dot(x_ref[...], w_ref[...], preferred_element_type=jnp.float32)


def _matmul(x, w):
    tm = 256
    n_rows = x.shape[0]
    return pl.pallas_call(
        _mm_body,
        grid=(n_rows // tm,),
        in_specs=[
            pl.BlockSpec((tm, _DM), lambda i: (i, 0)),
            pl.BlockSpec((_DM, _DM), lambda i: (0, 0)),
        ],
        out_specs=pl.BlockSpec((tm, _DM), lambda i: (i, 0)),
        out_shape=jax.ShapeDtypeStruct((n_rows, _DM), jnp.float32),
    )(x, w)


def _global_body(k_ref, v_ref, qg_ref, p_ref, pt_ref, o_ref):
    qrow = qg_ref[0]
    a = k_ref[0] * qrow
    logits = jnp.dot(a, p_ref[...], preferred_element_type=jnp.float32)
    m = jnp.max(logits, axis=0, keepdims=True)
    e = jnp.exp(logits - m)
    s = jnp.sum(e, axis=0, keepdims=True)
    prob = e / s
    pe = jnp.dot(prob, pt_ref[...], preferred_element_type=jnp.float32)
    o_ref[0, 0, :] = jnp.sum(pe * v_ref[0], axis=0)


def _global_attention(kt, vt, qg, p, pt):
    return pl.pallas_call(
        _global_body,
        grid=(4,),
        in_specs=[
            pl.BlockSpec((1, _N, _DM), lambda g: (g // 2, 0, 0)),
            pl.BlockSpec((1, _N, _DM), lambda g: (g // 2, 0, 0)),
            pl.BlockSpec((1, 1, _DM), lambda g: (g, 0, 0)),
            pl.BlockSpec((_DM, _H), lambda g: (0, 0)),
            pl.BlockSpec((_H, _DM), lambda g: (0, 0)),
        ],
        out_specs=pl.BlockSpec((1, 1, _DM), lambda g: (g, 0, 0)),
        out_shape=jax.ShapeDtypeStruct((4, 1, _DM), jnp.float32),
    )(kt, vt, qg, p, pt)


_PAIR = 2
_R = 8
_PR = _PAIR * _R
_WROWS = _CH + 16
_OH = _CH // 2


def _sc_body(qt_hbm, kt_hbm, vt_hbm, idx_hbm, out_hbm,
             idxv, qv, krand, vrand, kwin, vwin, kgv, vgv, outv,
             semk, semv, semw):
    wid = lax.axis_index("s") * _NC + lax.axis_index("c")

    npairs = _TPW // _PAIR

    gbase = pl.multiple_of((wid // _NS) * _N, _N)
    pltpu.sync_copy(kt_hbm.at[pl.ds(gbase, 1)], kgv)
    pltpu.sync_copy(vt_hbm.at[pl.ds(gbase, 1)], vgv)
    pltpu.sync_copy(idx_hbm.at[pl.ds(pl.multiple_of(wid * npairs, 8),
                                     npairs)], idxv)

    pltpu.async_copy(kt_hbm.at[idxv.at[0]], krand, semk)

    def chunk_body(c, carry):
        base = pl.multiple_of(wid * _TPW + c * _CH, _CH)
        pltpu.sync_copy(qt_hbm.at[pl.ds(base, _OH)], qv)

        wload = pl.multiple_of(
            jnp.clip(base - 8, 0, _NT - _WROWS), 8)
        woff = base - wload
        cw = pltpu.async_copy(kt_hbm.at[pl.ds(wload, _WROWS)], kwin, semw)
        cw2 = pltpu.async_copy(vt_hbm.at[pl.ds(wload, _WROWS)], vwin, semw)
        cw.wait()
        cw2.wait()

        def pair_body(p, carry2):
            gp = c * (_CH // _PAIR) + p
            pltpu.make_async_copy(kt_hbm.at[idxv.at[gp]], krand, semk).wait()
            cv = pltpu.async_copy(vt_hbm.at[idxv.at[gp]], vrand, semv)

            t0 = p * _PAIR
            tl0 = t0 - (p // (_OH // _PAIR)) * _OH
            wi0 = [jnp.clip(t0 + w - 1 + woff, 0, _WROWS - 1)
                   for w in range(3)]
            wi1 = [jnp.clip(t0 + w + woff, 0, _WROWS - 1) for w in range(3)]

            def d_body(d, accs):
                ds = pl.ds(d * 16, 16)
                new = list(accs)
                qd0 = qv[tl0, ds]
                qd1 = qv[tl0 + 1, ds]
                kg = kgv[0, ds]
                new[0] = new[0] + qd0 * kg
                new[_KK] = new[_KK] + qd1 * kg
                for r in range(_R):
                    new[1 + r] = new[1 + r] + qd0 * krand[r, ds]
                    new[_KK + 1 + r] = new[_KK + 1 + r] + qd1 * krand[_R + r, ds]
                for w in range(3):
                    new[9 + w] = new[9 + w] + qd0 * kwin[wi0[w], ds]
                    new[_KK + 9 + w] = new[_KK + 9 + w] + qd1 * kwin[wi1[w], ds]
                return tuple(new)

            zero = jnp.zeros((16,), jnp.float32)
            accs = lax.fori_loop(0, _DK, d_body, (zero,) * (2 * _KK))

            all_ws = []
            for u in range(_PAIR):
                ko = u * _KK
                m = accs[ko]
                for k in range(1, _KK):
                    m = jnp.maximum(m, accs[ko + k])
                es = tuple(jnp.exp(accs[ko + k] - m) for k in range(_KK))
                s = es[0]
                for k in range(1, _KK):
                    s = s + es[k]
                inv = 1.0 / s
                all_ws.append(tuple(e * inv for e in es))

            cv.wait()
            nxt = jnp.minimum(gp + 1, npairs - 1)
            pltpu.async_copy(kt_hbm.at[idxv.at[nxt]], krand, semk)

            ws0, ws1 = all_ws

            def o_body(d, carry3):
                ds = pl.ds(d * 16, 16)
                vg = vgv[0, ds]
                acc0 = ws0[0] * vg
                acc1 = ws1[0] * vg
                for r in range(_R):
                    acc0 = acc0 + ws0[1 + r] * vrand[r, ds]
                    acc1 = acc1 + ws1[1 + r] * vrand[_R + r, ds]
                for w in range(3):
                    acc0 = acc0 + ws0[9 + w] * vwin[wi0[w], ds]
                    acc1 = acc1 + ws1[9 + w] * vwin[wi1[w], ds]
                outv[tl0, ds] = acc0
                outv[tl0 + 1, ds] = acc1
                return carry3

            lax.fori_loop(0, _DK, o_body, 0)

            @pl.when(p == (_OH // _PAIR) - 1)
            def _():
                pltpu.sync_copy(outv, out_hbm.at[pl.ds(base, _OH)])
                pltpu.sync_copy(
                    qt_hbm.at[pl.ds(pl.multiple_of(base + _OH, 8), _OH)], qv)

            @pl.when(p == (_CH // _PAIR) - 1)
            def _():
                pltpu.sync_copy(
                    outv, out_hbm.at[pl.ds(
                        pl.multiple_of(base + _OH, 8), _OH)])

            return carry2

        lax.fori_loop(0, _CH // _PAIR, pair_body, 0)
        return carry

    lax.fori_loop(0, _NCH, chunk_body, 0)
    pltpu.make_async_copy(kt_hbm.at[idxv.at[npairs - 1]], krand, semk).wait()


def _sc_attention(qt, kt, vt, idx_rand):
    mesh = plsc.VectorSubcoreMesh(core_axis_name="c", subcore_axis_name="s")
    fn = functools.partial(
        pl.kernel,
        mesh=mesh,
        out_type=jax.ShapeDtypeStruct((_NT, _DM), jnp.float32),
        scratch_types=[
            pltpu.VMEM((_TPW // _PAIR, _PR), jnp.int32),
            pltpu.VMEM((_OH, _DM), jnp.float32),
            pltpu.VMEM((_PR, _DM), jnp.float32),
            pltpu.VMEM((_PR, _DM), jnp.float32),
            pltpu.VMEM((_WROWS, _DM), jnp.float32),
            pltpu.VMEM((_WROWS, _DM), jnp.float32),
            pltpu.VMEM((1, _DM), jnp.float32),
            pltpu.VMEM((1, _DM), jnp.float32),
            pltpu.VMEM((_OH, _DM), jnp.float32),
            pltpu.SemaphoreType.DMA,
            pltpu.SemaphoreType.DMA,
            pltpu.SemaphoreType.DMA,
        ],
    )(_sc_body)
    return fn(qt, kt, vt, idx_rand.reshape(_NT // _PAIR, _PR))


def _out_body(x_ref, g_ref, tgt_ref, w_ref, o_ref):
    x = x_ref[...]
    tgt = tgt_ref[pl.program_id(0)]
    rows = lax.broadcasted_iota(jnp.int32, x.shape, 0)
    x = jnp.where(rows == tgt, g_ref[0], x)
    o_ref[...] = jnp.dot(x, w_ref[...], preferred_element_type=jnp.float32)


def _out_matmul(x, g16, tgt, w):
    tm = 256
    return pl.pallas_call(
        _out_body,
        grid=(_NT // tm,),
        in_specs=[
            pl.BlockSpec((tm, _DM), lambda i: (i, 0)),
            pl.BlockSpec((1, 1, _DM), lambda i: (i, 0, 0)),
            pl.BlockSpec(memory_space=pltpu.SMEM),
            pl.BlockSpec((_DM, _DM), lambda i: (0, 0)),
        ],
        out_specs=pl.BlockSpec((tm, _DM), lambda i: (i, 0)),
        out_shape=jax.ShapeDtypeStruct((_NT, _DM), jnp.float32),
    )(x, g16, tgt, w)


def kernel(Q, K, V, Wq, Wk, Wv, Wo, idx):
    scale = 1.0 / math.sqrt(_DK)
    ct = jnp.arange(_DM)
    cols = (ct % _H) * _DK + ct // _H

    wq_in = jnp.transpose(Wq[cols, :] * scale)
    wk_in = jnp.transpose(Wk[cols, :])
    wv_in = jnp.transpose(Wv[cols, :])
    wo_in = jnp.transpose(Wo[:, cols])

    qt = _matmul(Q.reshape(_NT, _DM), wq_in)
    kt = _matmul(K.reshape(_NT, _DM), wk_in)
    vt = _matmul(V.reshape(_NT, _DM), wv_in)

    p = jax.nn.one_hot(ct % _H, _H, dtype=jnp.float32)
    pt = jnp.transpose(p)

    kt3 = kt.reshape(_B, _N, _DM)
    vt3 = vt.reshape(_B, _N, _DM)
    qg = qt.reshape(_B, _N, _DM)[:, jnp.array([0, _N - 1]), :].reshape(4, 1, _DM)
    out_g = _global_attention(kt3, vt3, qg, p, pt).reshape(_B, 2, _DM)

    idx32 = idx[:, 1:1 + _R].astype(jnp.int32)
    blocks = []
    for b in range(_B):
        pad = jnp.full((1, _R), b * _N, jnp.int32)
        blocks.append(jnp.concatenate([pad, idx32 + b * _N, pad], axis=0))
    idx_rand = jnp.concatenate(blocks, axis=0)

    sc_out = _sc_attention(qt, kt, vt, idx_rand)

    g16 = jnp.zeros((16, 1, _DM), jnp.float32)
    g16 = g16.at[0, 0].set(out_g[0, 0]).at[7, 0].set(out_g[0, 1])
    g16 = g16.at[8, 0].set(out_g[1, 0]).at[15, 0].set(out_g[1, 1])
    tgt = jnp.array([0] + [-1] * 6 + [255, 0] + [-1] * 6 + [255], jnp.int32)

    x = _out_matmul(sc_out, g16, tgt, wo_in)
    return x.reshape(_B, _N, _DM)


# speedup vs baseline: 1.3038x; 1.0405x over previous
import functools
import math

import jax
import jax.numpy as jnp
from jax import lax
from jax.experimental import pallas as pl
from jax.experimental.pallas import tpu as pltpu
from jax.experimental.pallas import tpu_sc as plsc

_B = 2
_N = 2048
_DM = 1024
_H = 16
_DK = 64
_KK = 12
_NT = _B * _N

_NC = 2
_NS = 16
_NWRK = _NC * _NS
_TPW = _NT // _NWRK
_CH = 16
_NCH = _TPW // _CH


def _proj_body(q_ref, k_ref, v_ref, wq_ref, wk_ref, wv_ref,
               oq_ref, ok_ref, ov_ref):
    oq_ref[...] = jnp.dot(q_ref[...], wq_ref[...],
                          preferred_element_type=jnp.float32)
    ok_ref[...] = jnp.dot(k_ref[...], wk_ref[...],
                          preferred_element_type=jnp.float32)
    ov_ref[...] = jnp.dot(v_ref[...], wv_ref[...],
                          preferred_element_type=jnp.float32)


def _projections(q, k, v, wq_in, wk_in, wv_in):
    tm = 256
    xspec = pl.BlockSpec((tm, _DM), lambda i: (i, 0))
    wspec = pl.BlockSpec((_DM, _DM), lambda i: (0, 0))
    return pl.pallas_call(
        _proj_body,
        grid=(_NT // tm,),
        in_specs=[xspec, xspec, xspec, wspec, wspec, wspec],
        out_specs=[xspec, xspec, xspec],
        out_shape=[jax.ShapeDtypeStruct((_NT, _DM), jnp.float32)] * 3,
    )(q, k, v, wq_in, wk_in, wv_in)


def _mm_body(x_ref, w_ref, o_ref):
    o_ref[...] = jnp.dot(x_ref[...], w_ref[...], preferred_element_type=jnp.float32)


def _matmul(x, w):
    tm = 256
    n_rows = x.shape[0]
    return pl.pallas_call(
        _mm_body,
        grid=(n_rows // tm,),
        in_specs=[
            pl.BlockSpec((tm, _DM), lambda i: (i, 0)),
            pl.BlockSpec((_DM, _DM), lambda i: (0, 0)),
        ],
        out_specs=pl.BlockSpec((tm, _DM), lambda i: (i, 0)),
        out_shape=jax.ShapeDtypeStruct((n_rows, _DM), jnp.float32),
    )(x, w)


def _global_body(k_ref, v_ref, qg_ref, p_ref, pt_ref, o_ref):
    qrow = qg_ref[0]
    a = k_ref[0] * qrow
    logits = jnp.dot(a, p_ref[...], preferred_element_type=jnp.float32)
    m = jnp.max(logits, axis=0, keepdims=True)
    e = jnp.exp(logits - m)
    s = jnp.sum(e, axis=0, keepdims=True)
    prob = e / s
    pe = jnp.dot(prob, pt_ref[...], preferred_element_type=jnp.float32)
    o_ref[0, 0, :] = jnp.sum(pe * v_ref[0], axis=0)


def _global_attention(kt, vt, qg, p, pt):
    return pl.pallas_call(
        _global_body,
        grid=(4,),
        in_specs=[
            pl.BlockSpec((1, _N, _DM), lambda g: (g // 2, 0, 0)),
            pl.BlockSpec((1, _N, _DM), lambda g: (g // 2, 0, 0)),
            pl.BlockSpec((1, 1, _DM), lambda g: (g, 0, 0)),
            pl.BlockSpec((_DM, _H), lambda g: (0, 0)),
            pl.BlockSpec((_H, _DM), lambda g: (0, 0)),
        ],
        out_specs=pl.BlockSpec((1, 1, _DM), lambda g: (g, 0, 0)),
        out_shape=jax.ShapeDtypeStruct((4, 1, _DM), jnp.float32),
    )(kt, vt, qg, p, pt)


_PAIR = 2
_R = 8
_PR = _PAIR * _R
_WROWS = _CH + 16
_OH = _CH // 2


def _sc_body(qt_hbm, kt_hbm, vt_hbm, idx_hbm, out_hbm,
             idxv, qv, krand, vrand, kwin, vwin, kgv, vgv, outv,
             semk, semv, semw):
    wid = lax.axis_index("s") * _NC + lax.axis_index("c")

    npairs = _TPW // _PAIR

    gbase = pl.multiple_of((wid // _NS) * _N, _N)
    pltpu.sync_copy(kt_hbm.at[pl.ds(gbase, 1)], kgv)
    pltpu.sync_copy(vt_hbm.at[pl.ds(gbase, 1)], vgv)
    pltpu.sync_copy(idx_hbm.at[pl.ds(pl.multiple_of(wid * npairs, 8),
                                     npairs)], idxv)

    pltpu.async_copy(kt_hbm.at[idxv.at[0]], krand, semk)

    def chunk_body(c, carry):
        base = pl.multiple_of(wid * _TPW + c * _CH, _CH)
        pltpu.sync_copy(qt_hbm.at[pl.ds(base, _OH)], qv)

        wload = pl.multiple_of(
            jnp.clip(base - 8, 0, _NT - _WROWS), 8)
        woff = base - wload
        cw = pltpu.async_copy(kt_hbm.at[pl.ds(wload, _WROWS)], kwin, semw)
        cw2 = pltpu.async_copy(vt_hbm.at[pl.ds(wload, _WROWS)], vwin, semw)
        cw.wait()
        cw2.wait()

        def pair_body(p, carry2):
            gp = c * (_CH // _PAIR) + p
            pltpu.make_async_copy(kt_hbm.at[idxv.at[gp]], krand, semk).wait()
            cv = pltpu.async_copy(vt_hbm.at[idxv.at[gp]], vrand, semv)

            t0 = p * _PAIR
            tl0 = t0 - (p // (_OH // _PAIR)) * _OH
            wi0 = [jnp.clip(t0 + w - 1 + woff, 0, _WROWS - 1)
                   for w in range(3)]
            wi1 = [jnp.clip(t0 + w + woff, 0, _WROWS - 1) for w in range(3)]

            def d_body(d, accs):
                ds = pl.ds(d * 16, 16)
                new = list(accs)
                qd0 = qv[tl0, ds]
                qd1 = qv[tl0 + 1, ds]
                kg = kgv[0, ds]
                new[0] = new[0] + qd0 * kg
                new[_KK] = new[_KK] + qd1 * kg
                for r in range(_R):
                    new[1 + r] = new[1 + r] + qd0 * krand[r, ds]
                    new[_KK + 1 + r] = new[_KK + 1 + r] + qd1 * krand[_R + r, ds]
                for w in range(3):
                    new[9 + w] = new[9 + w] + qd0 * kwin[wi0[w], ds]
                    new[_KK + 9 + w] = new[_KK + 9 + w] + qd1 * kwin[wi1[w], ds]
                return tuple(new)

            zero = jnp.zeros((16,), jnp.float32)
            accs = lax.fori_loop(0, _DK, d_body, (zero,) * (2 * _KK))

            all_ws = []
            for u in range(_PAIR):
                ko = u * _KK
                m = accs[ko]
                for k in range(1, _KK):
                    m = jnp.maximum(m, accs[ko + k])
                es = tuple(jnp.exp(accs[ko + k] - m) for k in range(_KK))
                s = es[0]
                for k in range(1, _KK):
                    s = s + es[k]
                inv = 1.0 / s
                all_ws.append(tuple(e * inv for e in es))

            cv.wait()
            nxt = jnp.minimum(gp + 1, npairs - 1)
            pltpu.async_copy(kt_hbm.at[idxv.at[nxt]], krand, semk)

            ws0, ws1 = all_ws

            def o_body(d, carry3):
                ds = pl.ds(d * 16, 16)
                vg = vgv[0, ds]
                acc0 = ws0[0] * vg
                acc1 = ws1[0] * vg
                for r in range(_R):
                    acc0 = acc0 + ws0[1 + r] * vrand[r, ds]
                    acc1 = acc1 + ws1[1 + r] * vrand[_R + r, ds]
                for w in range(3):
                    acc0 = acc0 + ws0[9 + w] * vwin[wi0[w], ds]
                    acc1 = acc1 + ws1[9 + w] * vwin[wi1[w], ds]
                outv[tl0, ds] = acc0
                outv[tl0 + 1, ds] = acc1
                return carry3

            lax.fori_loop(0, _DK, o_body, 0)

            @pl.when(p == (_OH // _PAIR) - 1)
            def _():
                pltpu.sync_copy(outv, out_hbm.at[pl.ds(base, _OH)])
                pltpu.sync_copy(
                    qt_hbm.at[pl.ds(pl.multiple_of(base + _OH, 8), _OH)], qv)

            @pl.when(p == (_CH // _PAIR) - 1)
            def _():
                pltpu.sync_copy(
                    outv, out_hbm.at[pl.ds(
                        pl.multiple_of(base + _OH, 8), _OH)])

            return carry2

        lax.fori_loop(0, _CH // _PAIR, pair_body, 0)
        return carry

    lax.fori_loop(0, _NCH, chunk_body, 0)
    pltpu.make_async_copy(kt_hbm.at[idxv.at[npairs - 1]], krand, semk).wait()


def _sc_attention(qt, kt, vt, idx_rand):
    mesh = plsc.VectorSubcoreMesh(core_axis_name="c", subcore_axis_name="s")
    fn = functools.partial(
        pl.kernel,
        mesh=mesh,
        out_type=jax.ShapeDtypeStruct((_NT, _DM), jnp.float32),
        scratch_types=[
            pltpu.VMEM((_TPW // _PAIR, _PR), jnp.int32),
            pltpu.VMEM((_OH, _DM), jnp.float32),
            pltpu.VMEM((_PR, _DM), jnp.float32),
            pltpu.VMEM((_PR, _DM), jnp.float32),
            pltpu.VMEM((_WROWS, _DM), jnp.float32),
            pltpu.VMEM((_WROWS, _DM), jnp.float32),
            pltpu.VMEM((1, _DM), jnp.float32),
            pltpu.VMEM((1, _DM), jnp.float32),
            pltpu.VMEM((_OH, _DM), jnp.float32),
            pltpu.SemaphoreType.DMA,
            pltpu.SemaphoreType.DMA,
            pltpu.SemaphoreType.DMA,
        ],
    )(_sc_body)
    return fn(qt, kt, vt, idx_rand.reshape(_NT // _PAIR, _PR))


def _out_body(x_ref, g_ref, tgt_ref, w_ref, o_ref):
    x = x_ref[...]
    tgt = tgt_ref[pl.program_id(0)]
    rows = lax.broadcasted_iota(jnp.int32, x.shape, 0)
    x = jnp.where(rows == tgt, g_ref[0], x)
    o_ref[...] = jnp.dot(x, w_ref[...], preferred_element_type=jnp.float32)


def _out_matmul(x, g16, tgt, w):
    tm = 256
    return pl.pallas_call(
        _out_body,
        grid=(_NT // tm,),
        in_specs=[
            pl.BlockSpec((tm, _DM), lambda i: (i, 0)),
            pl.BlockSpec((1, 1, _DM), lambda i: (i, 0, 0)),
            pl.BlockSpec(memory_space=pltpu.SMEM),
            pl.BlockSpec((_DM, _DM), lambda i: (0, 0)),
        ],
        out_specs=pl.BlockSpec((tm, _DM), lambda i: (i, 0)),
        out_shape=jax.ShapeDtypeStruct((_NT, _DM), jnp.float32),
    )(x, g16, tgt, w)


def kernel(Q, K, V, Wq, Wk, Wv, Wo, idx):
    scale = 1.0 / math.sqrt(_DK)
    ct = jnp.arange(_DM)
    cols = (ct % _H) * _DK + ct // _H

    wq_in = jnp.transpose(Wq[cols, :] * scale)
    wk_in = jnp.transpose(Wk[cols, :])
    wv_in = jnp.transpose(Wv[cols, :])
    wo_in = jnp.transpose(Wo[:, cols])

    qt, kt, vt = _projections(Q.reshape(_NT, _DM), K.reshape(_NT, _DM),
                              V.reshape(_NT, _DM), wq_in, wk_in, wv_in)

    p = jax.nn.one_hot(ct % _H, _H, dtype=jnp.float32)
    pt = jnp.transpose(p)

    kt3 = kt.reshape(_B, _N, _DM)
    vt3 = vt.reshape(_B, _N, _DM)
    qg = qt.reshape(_B, _N, _DM)[:, jnp.array([0, _N - 1]), :].reshape(4, 1, _DM)
    out_g = _global_attention(kt3, vt3, qg, p, pt).reshape(_B, 2, _DM)

    idx32 = idx[:, 1:1 + _R].astype(jnp.int32)
    blocks = []
    for b in range(_B):
        pad = jnp.full((1, _R), b * _N, jnp.int32)
        blocks.append(jnp.concatenate([pad, idx32 + b * _N, pad], axis=0))
    idx_rand = jnp.concatenate(blocks, axis=0)

    sc_out = _sc_attention(qt, kt, vt, idx_rand)

    g16 = jnp.zeros((16, 1, _DM), jnp.float32)
    g16 = g16.at[0, 0].set(out_g[0, 0]).at[7, 0].set(out_g[0, 1])
    g16 = g16.at[8, 0].set(out_g[1, 0]).at[15, 0].set(out_g[1, 1])
    tgt = jnp.array([0] + [-1] * 6 + [255, 0] + [-1] * 6 + [255], jnp.int32)

    x = _out_matmul(sc_out, g16, tgt, wo_in)
    return x.reshape(_B, _N, _DM)


# speedup vs baseline: 1.3787x; 1.0575x over previous
import functools
import math

import jax
import jax.numpy as jnp
from jax import lax
from jax.experimental import pallas as pl
from jax.experimental.pallas import tpu as pltpu
from jax.experimental.pallas import tpu_sc as plsc

_B = 2
_N = 2048
_DM = 1024
_H = 16
_DK = 64
_KK = 12
_NT = _B * _N

_NC = 2
_NS = 16
_NWRK = _NC * _NS
_TPW = _NT // _NWRK
_CH = 16
_NCH = _TPW // _CH


def _proj_body(q_ref, k_ref, v_ref, wq_ref, wk_ref, wv_ref,
               oq_ref, ok_ref, ov_ref):
    oq_ref[...] = jnp.dot(q_ref[...], wq_ref[...],
                          preferred_element_type=jnp.float32)
    ok_ref[...] = jnp.dot(k_ref[...], wk_ref[...],
                          preferred_element_type=jnp.float32)
    ov_ref[...] = jnp.dot(v_ref[...], wv_ref[...],
                          preferred_element_type=jnp.float32)


def _projections(q, k, v, wq_in, wk_in, wv_in):
    tm = 256
    xspec = pl.BlockSpec((tm, _DM), lambda i: (i, 0))
    wspec = pl.BlockSpec((_DM, _DM), lambda i: (0, 0))
    return pl.pallas_call(
        _proj_body,
        grid=(_NT // tm,),
        in_specs=[xspec, xspec, xspec, wspec, wspec, wspec],
        out_specs=[xspec, xspec, xspec],
        out_shape=[jax.ShapeDtypeStruct((_NT, _DM), jnp.float32)] * 3,
    )(q, k, v, wq_in, wk_in, wv_in)


def _mm_body(x_ref, w_ref, o_ref):
    o_ref[...] = jnp.dot(x_ref[...], w_ref[...], preferred_element_type=jnp.float32)


def _matmul(x, w):
    tm = 256
    n_rows = x.shape[0]
    return pl.pallas_call(
        _mm_body,
        grid=(n_rows // tm,),
        in_specs=[
            pl.BlockSpec((tm, _DM), lambda i: (i, 0)),
            pl.BlockSpec((_DM, _DM), lambda i: (0, 0)),
        ],
        out_specs=pl.BlockSpec((tm, _DM), lambda i: (i, 0)),
        out_shape=jax.ShapeDtypeStruct((n_rows, _DM), jnp.float32),
    )(x, w)


def _global_body(k_ref, v_ref, qg_ref, p_ref, pt_ref, o_ref):
    qrow = qg_ref[0]
    a = k_ref[0] * qrow
    logits = jnp.dot(a, p_ref[...], preferred_element_type=jnp.float32)
    m = jnp.max(logits, axis=0, keepdims=True)
    e = jnp.exp(logits - m)
    s = jnp.sum(e, axis=0, keepdims=True)
    prob = e / s
    pe = jnp.dot(prob, pt_ref[...], preferred_element_type=jnp.float32)
    o_ref[0, 0, :] = jnp.sum(pe * v_ref[0], axis=0)


def _global_attention(kt, vt, qg, p, pt):
    return pl.pallas_call(
        _global_body,
        grid=(4,),
        in_specs=[
            pl.BlockSpec((1, _N, _DM), lambda g: (g // 2, 0, 0)),
            pl.BlockSpec((1, _N, _DM), lambda g: (g // 2, 0, 0)),
            pl.BlockSpec((1, 1, _DM), lambda g: (g, 0, 0)),
            pl.BlockSpec((_DM, _H), lambda g: (0, 0)),
            pl.BlockSpec((_H, _DM), lambda g: (0, 0)),
        ],
        out_specs=pl.BlockSpec((1, 1, _DM), lambda g: (g, 0, 0)),
        out_shape=jax.ShapeDtypeStruct((4, 1, _DM), jnp.float32),
    )(kt, vt, qg, p, pt)


_PAIR = 2
_R = 8
_PR = _PAIR * _R
_WROWS = _CH + 16
_OH = _CH // 2


def _sc_body(qt_hbm, kt_hbm, vt_hbm, idx_hbm, out_hbm,
             idxv, qv, krand, vrand, kwin, vwin, kgv, vgv, outv,
             semk, semv, semw):
    wid = lax.axis_index("s") * _NC + lax.axis_index("c")

    npairs = _TPW // _PAIR

    gbase = pl.multiple_of((wid // _NS) * _N, _N)
    pltpu.sync_copy(kt_hbm.at[pl.ds(gbase, 1)], kgv)
    pltpu.sync_copy(vt_hbm.at[pl.ds(gbase, 1)], vgv)
    pltpu.sync_copy(idx_hbm.at[pl.ds(pl.multiple_of(wid * npairs, 8),
                                     npairs)], idxv)

    pltpu.async_copy(kt_hbm.at[idxv.at[0]], krand, semk)

    def chunk_body(c, carry):
        base = pl.multiple_of(wid * _TPW + c * _CH, _CH)
        pltpu.sync_copy(qt_hbm.at[pl.ds(base, _OH)], qv)

        wload = pl.multiple_of(
            jnp.clip(base - 8, 0, _NT - _WROWS), 8)
        woff = base - wload
        cw = pltpu.async_copy(kt_hbm.at[pl.ds(wload, _WROWS)], kwin, semw)
        pltpu.async_copy(vt_hbm.at[pl.ds(wload, _WROWS)], vwin, semw)
        cw.wait()

        def pair_body(p, carry2):
            gp = c * (_CH // _PAIR) + p
            pltpu.make_async_copy(kt_hbm.at[idxv.at[gp]], krand, semk).wait()
            cv = pltpu.async_copy(vt_hbm.at[idxv.at[gp]], vrand, semv)

            t0 = p * _PAIR
            tl0 = t0 - (p // (_OH // _PAIR)) * _OH
            wi0 = [jnp.clip(t0 + w - 1 + woff, 0, _WROWS - 1)
                   for w in range(3)]
            wi1 = [jnp.clip(t0 + w + woff, 0, _WROWS - 1) for w in range(3)]

            def d_body(d, accs):
                ds = pl.ds(d * 16, 16)
                new = list(accs)
                qd0 = qv[tl0, ds]
                qd1 = qv[tl0 + 1, ds]
                kg = kgv[0, ds]
                new[0] = new[0] + qd0 * kg
                new[_KK] = new[_KK] + qd1 * kg
                for r in range(_R):
                    new[1 + r] = new[1 + r] + qd0 * krand[r, ds]
                    new[_KK + 1 + r] = new[_KK + 1 + r] + qd1 * krand[_R + r, ds]
                for w in range(3):
                    new[9 + w] = new[9 + w] + qd0 * kwin[wi0[w], ds]
                    new[_KK + 9 + w] = new[_KK + 9 + w] + qd1 * kwin[wi1[w], ds]
                return tuple(new)

            zero = jnp.zeros((16,), jnp.float32)
            accs = lax.fori_loop(0, _DK, d_body, (zero,) * (2 * _KK))

            all_ws = []
            for u in range(_PAIR):
                ko = u * _KK
                m = accs[ko]
                for k in range(1, _KK):
                    m = jnp.maximum(m, accs[ko + k])
                es = tuple(jnp.exp(accs[ko + k] - m) for k in range(_KK))
                s = es[0]
                for k in range(1, _KK):
                    s = s + es[k]
                inv = 1.0 / s
                all_ws.append(tuple(e * inv for e in es))

            @pl.when(p == 0)
            def _():
                pltpu.make_async_copy(
                    vt_hbm.at[pl.ds(wload, _WROWS)], vwin, semw).wait()

            cv.wait()
            nxt = jnp.minimum(gp + 1, npairs - 1)
            pltpu.async_copy(kt_hbm.at[idxv.at[nxt]], krand, semk)

            ws0, ws1 = all_ws

            def o_body(d, carry3):
                ds = pl.ds(d * 16, 16)
                vg = vgv[0, ds]
                acc0 = ws0[0] * vg
                acc1 = ws1[0] * vg
                for r in range(_R):
                    acc0 = acc0 + ws0[1 + r] * vrand[r, ds]
                    acc1 = acc1 + ws1[1 + r] * vrand[_R + r, ds]
                for w in range(3):
                    acc0 = acc0 + ws0[9 + w] * vwin[wi0[w], ds]
                    acc1 = acc1 + ws1[9 + w] * vwin[wi1[w], ds]
                outv[tl0, ds] = acc0
                outv[tl0 + 1, ds] = acc1
                return carry3

            lax.fori_loop(0, _DK, o_body, 0)

            @pl.when(p == (_OH // _PAIR) - 1)
            def _():
                pltpu.sync_copy(outv, out_hbm.at[pl.ds(base, _OH)])
                pltpu.sync_copy(
                    qt_hbm.at[pl.ds(pl.multiple_of(base + _OH, 8), _OH)], qv)

            @pl.when(p == (_CH // _PAIR) - 1)
            def _():
                pltpu.sync_copy(
                    outv, out_hbm.at[pl.ds(
                        pl.multiple_of(base + _OH, 8), _OH)])

            return carry2

        lax.fori_loop(0, _CH // _PAIR, pair_body, 0)
        return carry

    lax.fori_loop(0, _NCH, chunk_body, 0)
    pltpu.make_async_copy(kt_hbm.at[idxv.at[npairs - 1]], krand, semk).wait()


def _sc_attention(qt, kt, vt, idx_rand):
    mesh = plsc.VectorSubcoreMesh(core_axis_name="c", subcore_axis_name="s")
    fn = functools.partial(
        pl.kernel,
        mesh=mesh,
        out_type=jax.ShapeDtypeStruct((_NT, _DM), jnp.float32),
        scratch_types=[
            pltpu.VMEM((_TPW // _PAIR, _PR), jnp.int32),
            pltpu.VMEM((_OH, _DM), jnp.float32),
            pltpu.VMEM((_PR, _DM), jnp.float32),
            pltpu.VMEM((_PR, _DM), jnp.float32),
            pltpu.VMEM((_WROWS, _DM), jnp.float32),
            pltpu.VMEM((_WROWS, _DM), jnp.float32),
            pltpu.VMEM((1, _DM), jnp.float32),
            pltpu.VMEM((1, _DM), jnp.float32),
            pltpu.VMEM((_OH, _DM), jnp.float32),
            pltpu.SemaphoreType.DMA,
            pltpu.SemaphoreType.DMA,
            pltpu.SemaphoreType.DMA,
        ],
    )(_sc_body)
    return fn(qt, kt, vt, idx_rand.reshape(_NT // _PAIR, _PR))


def _out_body(x_ref, g_ref, tgt_ref, w_ref, o_ref):
    x = x_ref[...]
    tgt = tgt_ref[pl.program_id(0)]
    rows = lax.broadcasted_iota(jnp.int32, x.shape, 0)
    x = jnp.where(rows == tgt, g_ref[0], x)
    o_ref[...] = jnp.dot(x, w_ref[...], preferred_element_type=jnp.float32)


def _out_matmul(x, g16, tgt, w):
    tm = 256
    return pl.pallas_call(
        _out_body,
        grid=(_NT // tm,),
        in_specs=[
            pl.BlockSpec((tm, _DM), lambda i: (i, 0)),
            pl.BlockSpec((1, 1, _DM), lambda i: (i, 0, 0)),
            pl.BlockSpec(memory_space=pltpu.SMEM),
            pl.BlockSpec((_DM, _DM), lambda i: (0, 0)),
        ],
        out_specs=pl.BlockSpec((tm, _DM), lambda i: (i, 0)),
        out_shape=jax.ShapeDtypeStruct((_NT, _DM), jnp.float32),
    )(x, g16, tgt, w)


def kernel(Q, K, V, Wq, Wk, Wv, Wo, idx):
    scale = 1.0 / math.sqrt(_DK)
    ct = jnp.arange(_DM)
    cols = (ct % _H) * _DK + ct // _H

    wq_in = jnp.transpose(Wq[cols, :] * scale)
    wk_in = jnp.transpose(Wk[cols, :])
    wv_in = jnp.transpose(Wv[cols, :])
    wo_in = jnp.transpose(Wo[:, cols])

    qt, kt, vt = _projections(Q.reshape(_NT, _DM), K.reshape(_NT, _DM),
                              V.reshape(_NT, _DM), wq_in, wk_in, wv_in)

    p = jax.nn.one_hot(ct % _H, _H, dtype=jnp.float32)
    pt = jnp.transpose(p)

    kt3 = kt.reshape(_B, _N, _DM)
    vt3 = vt.reshape(_B, _N, _DM)
    qt3 = qt.reshape(_B, _N, _DM)
    qg = jnp.concatenate([qt3[:, :1, :], qt3[:, _N - 1:, :]],
                         axis=1).reshape(4, 1, _DM)
    out_g = _global_attention(kt3, vt3, qg, p, pt).reshape(_B, 2, _DM)

    idx32 = idx[:, 1:1 + _R].astype(jnp.int32)
    blocks = []
    for b in range(_B):
        pad = jnp.full((1, _R), b * _N, jnp.int32)
        blocks.append(jnp.concatenate([pad, idx32 + b * _N, pad], axis=0))
    idx_rand = jnp.concatenate(blocks, axis=0)

    sc_out = _sc_attention(qt, kt, vt, idx_rand)

    g16 = jnp.zeros((16, 1, _DM), jnp.float32)
    g16 = g16.at[0, 0].set(out_g[0, 0]).at[7, 0].set(out_g[0, 1])
    g16 = g16.at[8, 0].set(out_g[1, 0]).at[15, 0].set(out_g[1, 1])
    tgt = jnp.array([0] + [-1] * 6 + [255, 0] + [-1] * 6 + [255], jnp.int32)

    x = _out_matmul(sc_out, g16, tgt, wo_in)
    return x.reshape(_B, _N, _DM)


# speedup vs baseline: 1.3801x; 1.0010x over previous
import functools
import math

import jax
import jax.numpy as jnp
from jax import lax
from jax.experimental import pallas as pl
from jax.experimental.pallas import tpu as pltpu
from jax.experimental.pallas import tpu_sc as plsc

_B = 2
_N = 2048
_DM = 1024
_H = 16
_DK = 64
_KK = 12
_NT = _B * _N

_NC = 2
_NS = 16
_NWRK = _NC * _NS
_TPW = _NT // _NWRK
_CH = 16
_NCH = _TPW // _CH


def _proj_body(q_ref, k_ref, v_ref, wq_ref, wk_ref, wv_ref,
               oq_ref, ok_ref, ov_ref):
    oq_ref[...] = jnp.dot(q_ref[...], wq_ref[...],
                          preferred_element_type=jnp.float32)
    ok_ref[...] = jnp.dot(k_ref[...], wk_ref[...],
                          preferred_element_type=jnp.float32)
    ov_ref[...] = jnp.dot(v_ref[...], wv_ref[...],
                          preferred_element_type=jnp.float32)


def _projections(q, k, v, wq_in, wk_in, wv_in):
    tm = 256
    xspec = pl.BlockSpec((tm, _DM), lambda i: (i, 0))
    wspec = pl.BlockSpec((_DM, _DM), lambda i: (0, 0))
    return pl.pallas_call(
        _proj_body,
        grid=(_NT // tm,),
        in_specs=[xspec, xspec, xspec, wspec, wspec, wspec],
        out_specs=[xspec, xspec, xspec],
        out_shape=[jax.ShapeDtypeStruct((_NT, _DM), jnp.float32)] * 3,
    )(q, k, v, wq_in, wk_in, wv_in)


def _mm_body(x_ref, w_ref, o_ref):
    o_ref[...] = jnp.dot(x_ref[...], w_ref[...], preferred_element_type=jnp.float32)


def _matmul(x, w):
    tm = 256
    n_rows = x.shape[0]
    return pl.pallas_call(
        _mm_body,
        grid=(n_rows // tm,),
        in_specs=[
            pl.BlockSpec((tm, _DM), lambda i: (i, 0)),
            pl.BlockSpec((_DM, _DM), lambda i: (0, 0)),
        ],
        out_specs=pl.BlockSpec((tm, _DM), lambda i: (i, 0)),
        out_shape=jax.ShapeDtypeStruct((n_rows, _DM), jnp.float32),
    )(x, w)


def _global_body(k_ref, v_ref, qg_ref, p_ref, pt_ref, o_ref):
    qrow = qg_ref[0]
    a = k_ref[0] * qrow
    logits = jnp.dot(a, p_ref[...], preferred_element_type=jnp.float32)
    m = jnp.max(logits, axis=0, keepdims=True)
    e = jnp.exp(logits - m)
    s = jnp.sum(e, axis=0, keepdims=True)
    prob = e / s
    pe = jnp.dot(prob, pt_ref[...], preferred_element_type=jnp.float32)
    o_ref[0, 0, :] = jnp.sum(pe * v_ref[0], axis=0)


def _global_attention(kt, vt, qg, p, pt):
    return pl.pallas_call(
        _global_body,
        grid=(4,),
        in_specs=[
            pl.BlockSpec((1, _N, _DM), lambda g: (g // 2, 0, 0)),
            pl.BlockSpec((1, _N, _DM), lambda g: (g // 2, 0, 0)),
            pl.BlockSpec((1, 1, _DM), lambda g: (g, 0, 0)),
            pl.BlockSpec((_DM, _H), lambda g: (0, 0)),
            pl.BlockSpec((_H, _DM), lambda g: (0, 0)),
        ],
        out_specs=pl.BlockSpec((1, 1, _DM), lambda g: (g, 0, 0)),
        out_shape=jax.ShapeDtypeStruct((4, 1, _DM), jnp.float32),
    )(kt, vt, qg, p, pt)


_PAIR = 2
_R = 8
_PR = _PAIR * _R
_WROWS = _CH + 16
_OH = _CH // 2


def _sc_body(qt_hbm, kt_hbm, vt_hbm, idx_hbm, out_hbm,
             idxv, qv, krand, vrand, kwin, vwin, kgv, vgv, outv,
             semk, semv, semw):
    wid = lax.axis_index("s") * _NC + lax.axis_index("c")

    npairs = _TPW // _PAIR

    gbase = pl.multiple_of((wid // _NS) * _N, _N)
    pltpu.sync_copy(kt_hbm.at[pl.ds(gbase, 1)], kgv)
    pltpu.sync_copy(vt_hbm.at[pl.ds(gbase, 1)], vgv)
    pltpu.sync_copy(idx_hbm.at[pl.ds(pl.multiple_of(wid * npairs, 8),
                                     npairs)], idxv)

    pltpu.async_copy(kt_hbm.at[idxv.at[0]], krand, semk)

    def chunk_body(c, carry):
        base = pl.multiple_of(wid * _TPW + c * _CH, _CH)
        pltpu.sync_copy(qt_hbm.at[pl.ds(base, _OH)], qv)

        wload = pl.multiple_of(
            jnp.clip(base - 8, 0, _NT - _WROWS), 8)
        woff = base - wload
        cw = pltpu.async_copy(kt_hbm.at[pl.ds(wload, _WROWS)], kwin, semw)
        pltpu.async_copy(vt_hbm.at[pl.ds(wload, _WROWS)], vwin, semw)
        cw.wait()

        def pair_body(p, carry2):
            gp = c * (_CH // _PAIR) + p
            pltpu.make_async_copy(kt_hbm.at[idxv.at[gp]], krand, semk).wait()
            cv = pltpu.async_copy(vt_hbm.at[idxv.at[gp]], vrand, semv)

            t0 = p * _PAIR
            tl0 = t0 - (p // (_OH // _PAIR)) * _OH
            wi0 = [jnp.clip(t0 + w - 1 + woff, 0, _WROWS - 1)
                   for w in range(3)]
            wi1 = [jnp.clip(t0 + w + woff, 0, _WROWS - 1) for w in range(3)]

            def d_body(d2, accs):
                new = list(accs)
                for h in range(2):
                    ds = pl.ds(d2 * 32 + h * 16, 16)
                    qd0 = qv[tl0, ds]
                    qd1 = qv[tl0 + 1, ds]
                    kg = kgv[0, ds]
                    new[0] = new[0] + qd0 * kg
                    new[_KK] = new[_KK] + qd1 * kg
                    for r in range(_R):
                        new[1 + r] = new[1 + r] + qd0 * krand[r, ds]
                        new[_KK + 1 + r] = (new[_KK + 1 + r]
                                            + qd1 * krand[_R + r, ds])
                    for w in range(3):
                        new[9 + w] = new[9 + w] + qd0 * kwin[wi0[w], ds]
                        new[_KK + 9 + w] = (new[_KK + 9 + w]
                                            + qd1 * kwin[wi1[w], ds])
                return tuple(new)

            zero = jnp.zeros((16,), jnp.float32)
            accs = lax.fori_loop(0, _DK // 2, d_body, (zero,) * (2 * _KK))

            all_ws = []
            for u in range(_PAIR):
                ko = u * _KK
                m = accs[ko]
                for k in range(1, _KK):
                    m = jnp.maximum(m, accs[ko + k])
                es = tuple(jnp.exp(accs[ko + k] - m) for k in range(_KK))
                s = es[0]
                for k in range(1, _KK):
                    s = s + es[k]
                inv = 1.0 / s
                all_ws.append(tuple(e * inv for e in es))

            @pl.when(p == 0)
            def _():
                pltpu.make_async_copy(
                    vt_hbm.at[pl.ds(wload, _WROWS)], vwin, semw).wait()

            cv.wait()
            nxt = jnp.minimum(gp + 1, npairs - 1)
            pltpu.async_copy(kt_hbm.at[idxv.at[nxt]], krand, semk)

            ws0, ws1 = all_ws

            def o_body(d2, carry3):
                for h in range(2):
                    ds = pl.ds(d2 * 32 + h * 16, 16)
                    vg = vgv[0, ds]
                    acc0 = ws0[0] * vg
                    acc1 = ws1[0] * vg
                    for r in range(_R):
                        acc0 = acc0 + ws0[1 + r] * vrand[r, ds]
                        acc1 = acc1 + ws1[1 + r] * vrand[_R + r, ds]
                    for w in range(3):
                        acc0 = acc0 + ws0[9 + w] * vwin[wi0[w], ds]
                        acc1 = acc1 + ws1[9 + w] * vwin[wi1[w], ds]
                    outv[tl0, ds] = acc0
                    outv[tl0 + 1, ds] = acc1
                return carry3

            lax.fori_loop(0, _DK // 2, o_body, 0)

            @pl.when(p == (_OH // _PAIR) - 1)
            def _():
                pltpu.sync_copy(outv, out_hbm.at[pl.ds(base, _OH)])
                pltpu.sync_copy(
                    qt_hbm.at[pl.ds(pl.multiple_of(base + _OH, 8), _OH)], qv)

            @pl.when(p == (_CH // _PAIR) - 1)
            def _():
                pltpu.sync_copy(
                    outv, out_hbm.at[pl.ds(
                        pl.multiple_of(base + _OH, 8), _OH)])

            return carry2

        lax.fori_loop(0, _CH // _PAIR, pair_body, 0)
        return carry

    lax.fori_loop(0, _NCH, chunk_body, 0)
    pltpu.make_async_copy(kt_hbm.at[idxv.at[npairs - 1]], krand, semk).wait()


def _sc_attention(qt, kt, vt, idx_rand):
    mesh = plsc.VectorSubcoreMesh(core_axis_name="c", subcore_axis_name="s")
    fn = functools.partial(
        pl.kernel,
        mesh=mesh,
        out_type=jax.ShapeDtypeStruct((_NT, _DM), jnp.float32),
        scratch_types=[
            pltpu.VMEM((_TPW // _PAIR, _PR), jnp.int32),
            pltpu.VMEM((_OH, _DM), jnp.float32),
            pltpu.VMEM((_PR, _DM), jnp.float32),
            pltpu.VMEM((_PR, _DM), jnp.float32),
            pltpu.VMEM((_WROWS, _DM), jnp.float32),
            pltpu.VMEM((_WROWS, _DM), jnp.float32),
            pltpu.VMEM((1, _DM), jnp.float32),
            pltpu.VMEM((1, _DM), jnp.float32),
            pltpu.VMEM((_OH, _DM), jnp.float32),
            pltpu.SemaphoreType.DMA,
            pltpu.SemaphoreType.DMA,
            pltpu.SemaphoreType.DMA,
        ],
    )(_sc_body)
    return fn(qt, kt, vt, idx_rand.reshape(_NT // _PAIR, _PR))


def _out_body(x_ref, g_ref, tgt_ref, w_ref, o_ref):
    x = x_ref[...]
    tgt = tgt_ref[pl.program_id(0)]
    rows = lax.broadcasted_iota(jnp.int32, x.shape, 0)
    x = jnp.where(rows == tgt, g_ref[0], x)
    o_ref[...] = jnp.dot(x, w_ref[...], preferred_element_type=jnp.float32)


def _out_matmul(x, g16, tgt, w):
    tm = 256
    return pl.pallas_call(
        _out_body,
        grid=(_NT // tm,),
        in_specs=[
            pl.BlockSpec((tm, _DM), lambda i: (i, 0)),
            pl.BlockSpec((1, 1, _DM), lambda i: (i, 0, 0)),
            pl.BlockSpec(memory_space=pltpu.SMEM),
            pl.BlockSpec((_DM, _DM), lambda i: (0, 0)),
        ],
        out_specs=pl.BlockSpec((tm, _DM), lambda i: (i, 0)),
        out_shape=jax.ShapeDtypeStruct((_NT, _DM), jnp.float32),
    )(x, g16, tgt, w)


def kernel(Q, K, V, Wq, Wk, Wv, Wo, idx):
    scale = 1.0 / math.sqrt(_DK)
    ct = jnp.arange(_DM)
    cols = (ct % _H) * _DK + ct // _H

    wq_in = jnp.transpose(Wq[cols, :] * scale)
    wk_in = jnp.transpose(Wk[cols, :])
    wv_in = jnp.transpose(Wv[cols, :])
    wo_in = jnp.transpose(Wo[:, cols])

    qt, kt, vt = _projections(Q.reshape(_NT, _DM), K.reshape(_NT, _DM),
                              V.reshape(_NT, _DM), wq_in, wk_in, wv_in)

    p = jax.nn.one_hot(ct % _H, _H, dtype=jnp.float32)
    pt = jnp.transpose(p)

    kt3 = kt.reshape(_B, _N, _DM)
    vt3 = vt.reshape(_B, _N, _DM)
    qt3 = qt.reshape(_B, _N, _DM)
    qg = jnp.concatenate([qt3[:, :1, :], qt3[:, _N - 1:, :]],
                         axis=1).reshape(4, 1, _DM)
    out_g = _global_attention(kt3, vt3, qg, p, pt).reshape(_B, 2, _DM)

    idx32 = idx[:, 1:1 + _R].astype(jnp.int32)
    blocks = []
    for b in range(_B):
        pad = jnp.full((1, _R), b * _N, jnp.int32)
        blocks.append(jnp.concatenate([pad, idx32 + b * _N, pad], axis=0))
    idx_rand = jnp.concatenate(blocks, axis=0)

    sc_out = _sc_attention(qt, kt, vt, idx_rand)

    g16 = jnp.zeros((16, 1, _DM), jnp.float32)
    g16 = g16.at[0, 0].set(out_g[0, 0]).at[7, 0].set(out_g[0, 1])
    g16 = g16.at[8, 0].set(out_g[1, 0]).at[15, 0].set(out_g[1, 1])
    tgt = jnp.array([0] + [-1] * 6 + [255, 0] + [-1] * 6 + [255], jnp.int32)

    x = _out_matmul(sc_out, g16, tgt, wo_in)
    return x.reshape(_B, _N, _DM)
